# initial kernel scaffold (unmeasured)
import jax
import jax.numpy as jnp
from jax import lax
from jax.experimental import pallas as pl
from jax.experimental.pallas import tpu as pltpu


def kernel(
    x,
):
    def body(*refs):
        pass

    out_shape = jax.ShapeDtypeStruct(..., jnp.float32)
    return pl.pallas_call(body, out_shape=out_shape)(...)


# baseline (device time: 1066643 ns/iter reference)
import jax
import jax.numpy as jnp
from jax import lax
from jax.experimental import pallas as pl
from jax.experimental.pallas import tpu as pltpu


def kernel(x):
    m, n = x.shape
    half = m // 2

    def body(x_ref, out_ref, local_sem, send_x, recv_x, send_y, recv_y):
        my_x = lax.axis_index("x")
        my_y = lax.axis_index("y")
        x_peer = (1 - my_x, my_y)
        y_peer = (my_x, 1 - my_y)

        barrier = pltpu.get_barrier_semaphore()
        for nbr in (x_peer, y_peer):
            pl.semaphore_signal(
                barrier, inc=1, device_id=nbr,
                device_id_type=pl.DeviceIdType.MESH,
            )
        pl.semaphore_wait(barrier, 2)

        local = pltpu.make_async_copy(
            x_ref, out_ref.at[pl.ds(my_x * m, m)], local_sem
        )
        local.start()

        rdma_x = pltpu.make_async_remote_copy(
            src_ref=x_ref.at[pl.ds(my_y * half, half)],
            dst_ref=out_ref.at[pl.ds(my_x * m + my_y * half, half)],
            send_sem=send_x,
            recv_sem=recv_x,
            device_id=x_peer,
            device_id_type=pl.DeviceIdType.MESH,
        )
        rdma_x.start()
        rdma_x.wait()

        fwd_off = (1 - my_x) * m + my_y * half
        rdma_y = pltpu.make_async_remote_copy(
            src_ref=out_ref.at[pl.ds(fwd_off, half)],
            dst_ref=out_ref.at[pl.ds(fwd_off, half)],
            send_sem=send_y,
            recv_sem=recv_y,
            device_id=y_peer,
            device_id_type=pl.DeviceIdType.MESH,
        )
        rdma_y.start()
        rdma_y.wait()
        local.wait()

    return pl.pallas_call(
        body,
        out_shape=jax.ShapeDtypeStruct((2 * m, n), x.dtype),
        in_specs=[pl.BlockSpec(memory_space=pl.ANY)],
        out_specs=pl.BlockSpec(memory_space=pl.ANY),
        scratch_shapes=[
            pltpu.SemaphoreType.DMA,
            pltpu.SemaphoreType.DMA,
            pltpu.SemaphoreType.DMA,
            pltpu.SemaphoreType.DMA,
            pltpu.SemaphoreType.DMA,
        ],
        compiler_params=pltpu.CompilerParams(collective_id=0),
    )(x)


# device time: 254236 ns/iter; 4.1955x vs baseline; 4.1955x over previous
import jax
import jax.numpy as jnp
from jax import lax
from jax.experimental import pallas as pl
from jax.experimental.pallas import tpu as pltpu

K = 16
KL = 4


def kernel(x):
    m, n = x.shape
    half = m // 2
    ch = half // K
    chl = m // KL

    def body(x_ref, out_ref, stage, send_x, recv_x, send_y, recv_y,
             load_sems, store_sems):
        my_x = lax.axis_index("x")
        my_y = lax.axis_index("y")
        x_peer = (1 - my_x, my_y)
        y_peer = (my_x, 1 - my_y)

        barrier = pltpu.get_barrier_semaphore()
        for nbr in (x_peer, y_peer):
            pl.semaphore_signal(
                barrier, inc=1, device_id=nbr,
                device_id_type=pl.DeviceIdType.MESH,
            )
        pl.semaphore_wait(barrier, 2)

        send_base = my_y * half
        dst_base = my_x * m + my_y * half
        xr = []
        for c in range(K):
            r = pltpu.make_async_remote_copy(
                src_ref=x_ref.at[pl.ds(send_base + c * ch, ch)],
                dst_ref=out_ref.at[pl.ds(dst_base + c * ch, ch)],
                send_sem=send_x.at[c],
                recv_sem=recv_x.at[c],
                device_id=x_peer,
                device_id_type=pl.DeviceIdType.MESH,
            )
            r.start()
            xr.append(r)

        prev_store = [None, None]
        for c in range(KL):
            s = c % 2
            if prev_store[s] is not None:
                prev_store[s].wait()
            ld = pltpu.make_async_copy(
                x_ref.at[pl.ds(c * chl, chl)], stage.at[s], load_sems.at[s]
            )
            ld.start()
            ld.wait()
            st = pltpu.make_async_copy(
                stage.at[s], out_ref.at[pl.ds(my_x * m + c * chl, chl)],
                store_sems.at[s],
            )
            st.start()
            prev_store[s] = st

        fwd_base = (1 - my_x) * m + my_y * half
        yr = []
        for c in range(K):
            xr[c].wait_recv()
            r = pltpu.make_async_remote_copy(
                src_ref=out_ref.at[pl.ds(fwd_base + c * ch, ch)],
                dst_ref=out_ref.at[pl.ds(fwd_base + c * ch, ch)],
                send_sem=send_y.at[c],
                recv_sem=recv_y.at[c],
                device_id=y_peer,
                device_id_type=pl.DeviceIdType.MESH,
            )
            r.start()
            yr.append(r)

        for c in range(K):
            xr[c].wait_send()
            yr[c].wait()
        for st in prev_store:
            st.wait()

    return pl.pallas_call(
        body,
        out_shape=jax.ShapeDtypeStruct((2 * m, n), x.dtype),
        in_specs=[pl.BlockSpec(memory_space=pl.ANY)],
        out_specs=pl.BlockSpec(memory_space=pl.ANY),
        scratch_shapes=[
            pltpu.VMEM((2, chl, n), x.dtype),
            pltpu.SemaphoreType.DMA((K,)),
            pltpu.SemaphoreType.DMA((K,)),
            pltpu.SemaphoreType.DMA((K,)),
            pltpu.SemaphoreType.DMA((K,)),
            pltpu.SemaphoreType.DMA((2,)),
            pltpu.SemaphoreType.DMA((2,)),
        ],
        compiler_params=pltpu.CompilerParams(collective_id=0),
    )(x)
